# baseline (speedup 1.0000x reference)
import functools
import jax
import jax.numpy as jnp
from jax import lax
from jax.experimental import pallas as pl
from jax.experimental.pallas import tpu as pltpu
from jax.experimental.pallas import tpu_sc as plsc

NC = 2
NS = 16
L = 16
NW = NC * NS
R = 16

S_BLK = 2048
TC_B = 3


def _tc_add(x_ref, pos_ref, o_ref):
    o_ref[...] = x_ref[...] + pos_ref[...]


def _tc_call(x, pos_table):
    batch, seq_len, d_model = x.shape
    n_s = seq_len // S_BLK
    return pl.pallas_call(
        _tc_add,
        grid=(n_s, TC_B),
        in_specs=[
            pl.BlockSpec((1, S_BLK, d_model), lambda s, b: (b, s, 0)),
            pl.BlockSpec((S_BLK, d_model), lambda s, b: (s, 0)),
        ],
        out_specs=pl.BlockSpec((1, S_BLK, d_model), lambda s, b: (b, s, 0)),
        out_shape=jax.ShapeDtypeStruct((TC_B, seq_len, d_model), x.dtype),
    )(x, pos_table)


def _sc_add(x_hbm, pos_hbm, out_hbm, xA, pA, xB, pB, slA, slB, ssA, ssB):
    seq = pos_hbm.shape[0]
    d = pos_hbm.shape[1]
    pos_rows = seq // NW
    nchunks = pos_rows // R
    nvec = d // L
    wid = lax.axis_index("s") * NC + lax.axis_index("c")
    pos_base = wid * pos_rows
    xrow0 = pos_base

    def issue(chunk, xb, pb, sl):
        r0 = chunk * R
        pltpu.async_copy(x_hbm.at[pl.ds(xrow0 + r0, R)], xb, sl)
        pltpu.async_copy(pos_hbm.at[pl.ds(pos_base + r0, R)], pb, sl)

    def wait_loads(xb, pb, sl):
        pltpu.make_async_copy(x_hbm.at[pl.ds(0, R)], xb, sl).wait()
        pltpu.make_async_copy(pos_hbm.at[pl.ds(0, R)], pb, sl).wait()

    def drain_store(xb, ss):
        pltpu.make_async_copy(x_hbm.at[pl.ds(0, R)], xb, ss).wait()

    def compute(xb, pb):
        def row(r, carry):
            for k in range(nvec):
                sl_ = pl.ds(k * L, L)
                xb[r, sl_] = xb[r, sl_] + pb[r, sl_]
            return carry
        lax.fori_loop(0, R, row, 0)

    def store(chunk, xb, ss):
        pltpu.async_copy(xb, out_hbm.at[pl.ds(pos_base + chunk * R, R)], ss)

    issue(0, xA, pA, slA)

    def body(i, carry):
        cA = 2 * i
        cB = 2 * i + 1

        wait_loads(xA, pA, slA)
        compute(xA, pA)
        store(cA, xA, ssA)

        @pl.when(i > 0)
        def _():
            drain_store(xB, ssB)
        issue(cB, xB, pB, slB)

        wait_loads(xB, pB, slB)
        compute(xB, pB)
        store(cB, xB, ssB)

        drain_store(xA, ssA)

        @pl.when(i < nchunks // 2 - 1)
        def _():
            issue(cA + 2, xA, pA, slA)
        return carry

    lax.fori_loop(0, nchunks // 2, body, 0)
    drain_store(xB, ssB)


def _sc_call(x_b, pos_table):
    s, d = x_b.shape
    mesh = plsc.VectorSubcoreMesh(core_axis_name="c", subcore_axis_name="s")
    run = functools.partial(
        pl.kernel,
        mesh=mesh,
        out_type=jax.ShapeDtypeStruct((s, d), jnp.float32),
        scratch_types=(
            [pltpu.VMEM((R, d), jnp.float32)] * 4
            + [pltpu.SemaphoreType.DMA] * 4
        ),
    )(_sc_add)
    return run(x_b, pos_table)


def kernel(x, pos_table):
    out_sc = _sc_call(x[TC_B], pos_table)
    out_tc = _tc_call(x, pos_table)
    return jnp.concatenate([out_tc, out_sc[None]], axis=0)


# speedup vs baseline: 2.4478x; 2.4478x over previous
import jax
import jax.numpy as jnp
from jax.experimental import pallas as pl

S_BLK = 2048


def _add_kernel(x_ref, pos_ref, o_ref):
    o_ref[...] = x_ref[...] + pos_ref[...]


def kernel(x, pos_table):
    batch, seq_len, d_model = x.shape
    n_s = seq_len // S_BLK
    return pl.pallas_call(
        _add_kernel,
        grid=(n_s, batch),
        in_specs=[
            pl.BlockSpec((1, S_BLK, d_model), lambda s, b: (b, s, 0)),
            pl.BlockSpec((S_BLK, d_model), lambda s, b: (s, 0)),
        ],
        out_specs=pl.BlockSpec((1, S_BLK, d_model), lambda s, b: (b, s, 0)),
        out_shape=jax.ShapeDtypeStruct((batch, seq_len, d_model), x.dtype),
    )(x, pos_table)
